# baseline (speedup 1.0000x reference)
import functools

import jax
import jax.numpy as jnp
from jax import lax
from jax.experimental import pallas as pl
from jax.experimental.pallas import tpu as pltpu
from jax.experimental.pallas import tpu_sc as plsc

N = 10000
D = 128
E_MSG = 256000
E_TGT = 64000
C = 64
NTILES = 16
NPAD = 10240
ROWS_PER_TILE = NPAD // NTILES

_mesh = plsc.VectorSubcoreMesh(core_axis_name="c", subcore_axis_name="s")


def _sc_body(msg_src, msg_dst, tgt_src, tgt_dst, tw, x, agg_out,
             src_v, dst_v, rows_v, w_v, agg_sh, sem_i, sem_g, sem_s):
    c = lax.axis_index("c")
    s = lax.axis_index("s")
    w = s * 2 + c

    def zrow(i, carry):
        for k in range(D // 16):
            rows_v[0, i, pl.ds(k * 16, 16)] = jnp.zeros((16,), jnp.float32)
        return carry
    lax.fori_loop(0, C, zrow, 0)
    for k in range(ROWS_PER_TILE // C):
        pltpu.sync_copy(rows_v.at[0],
                        agg_sh.at[pl.ds(s * ROWS_PER_TILE + k * C, C)])
    plsc.subcore_barrier()

    def run_seg(srcs, dsts, n, base, weighted):
        def idx_descs(j):
            b = (base + j) * C
            slot = j % 8
            ds = [pltpu.make_async_copy(srcs.at[pl.ds(b, C)],
                                        src_v.at[slot], sem_i.at[slot]),
                  pltpu.make_async_copy(dsts.at[pl.ds(b, C)],
                                        dst_v.at[slot], sem_i.at[slot])]
            if weighted:
                ds.append(pltpu.make_async_copy(tw.at[pl.ds(b, C)],
                                                w_v.at[slot], sem_i.at[slot]))
            return ds

        def gather_desc(j):
            return pltpu.make_async_copy(x.at[src_v.at[j % 8]],
                                         rows_v.at[j % 5], sem_g.at[j % 5])

        def scatter_desc(j):
            return pltpu.make_async_copy(rows_v.at[j % 5],
                                         agg_sh.at[dst_v.at[j % 8]],
                                         sem_s.at[j % 4])

        def scale(j):
            rbuf = j % 5
            wbuf = j % 8

            def scale_body(e, carry):
                wv = w_v[wbuf, pl.ds((e // 16) * 16, 16)]
                oh = lax.iota(jnp.int32, 16) == (e % 16)
                ws = jnp.broadcast_to(jnp.sum(jnp.where(oh, wv, 0.0)), (16,))
                for k in range(D // 16):
                    rows_v[rbuf, e, pl.ds(k * 16, 16)] = (
                        rows_v[rbuf, e, pl.ds(k * 16, 16)] * ws)
                return carry
            lax.fori_loop(0, C, scale_body, 0)

        for k in range(5):
            for d in idx_descs(k):
                d.start()
        for k in range(3):
            for d in idx_descs(k):
                d.wait()
            gather_desc(k).start()

        def body(j, carry):
            gather_desc(j).wait()

            @pl.when(j >= 2)
            def _():
                scatter_desc(j - 2).wait()

            @pl.when(j + 3 < n)
            def _():
                for d in idx_descs(j + 3):
                    d.wait()
                gather_desc(j + 3).start()

            @pl.when(j + 5 < n)
            def _():
                for d in idx_descs(j + 5):
                    d.start()

            if weighted:
                scale(j)
            scatter_desc(j).start(add=True)
            return carry

        lax.fori_loop(0, n, body, 0)
        scatter_desc(n - 2).wait()
        scatter_desc(n - 1).wait()

    run_seg(msg_src, msg_dst, 125, w * 125, weighted=False)
    run_seg(tgt_src, tgt_dst, jnp.where(w < 8, 32, 31),
            w * 31 + jnp.minimum(w, 8), weighted=True)

    plsc.subcore_barrier()
    pltpu.sync_copy(agg_sh.at[pl.ds(s * ROWS_PER_TILE, ROWS_PER_TILE)],
                    agg_out.at[c, pl.ds(s * ROWS_PER_TILE, ROWS_PER_TILE)])


_sc_scatter = functools.partial(
    pl.kernel,
    out_type=jax.ShapeDtypeStruct((2, NPAD, D), jnp.float32),
    mesh=_mesh,
    compiler_params=pltpu.CompilerParams(needs_layout_passes=False),
    scratch_types=[
        pltpu.VMEM((8, C), jnp.int32),
        pltpu.VMEM((8, C), jnp.int32),
        pltpu.VMEM((5, C, D), jnp.float32),
        pltpu.VMEM((8, C), jnp.float32),
        pltpu.VMEM_SHARED((NPAD, D), jnp.float32),
        pltpu.SemaphoreType.DMA((8,)),
        pltpu.SemaphoreType.DMA((5,)),
        pltpu.SemaphoreType.DMA((4,)),
    ],
)(_sc_body)


ROWS_PER_BLK = 1000


def _root_body(x_ref, wroot_ref, brel_ref, root_ref):
    root_ref[...] = jnp.dot(x_ref[...], wroot_ref[...],
                            preferred_element_type=jnp.float32) + brel_ref[...]


_root = pl.pallas_call(
    _root_body,
    grid=(N // ROWS_PER_BLK,),
    in_specs=[
        pl.BlockSpec((ROWS_PER_BLK, D), lambda i: (i, 0)),
        pl.BlockSpec((D, D), lambda i: (0, 0)),
        pl.BlockSpec((1, D), lambda i: (0, 0)),
    ],
    out_specs=pl.BlockSpec((ROWS_PER_BLK, D), lambda i: (i, 0)),
    out_shape=jax.ShapeDtypeStruct((N, D), jnp.float32),
)


def _dense_body(agg_ref, root_ref, wrel_ref, wmu_ref, bmu_ref, out_ref):
    agg = agg_ref[0] + agg_ref[1]
    conv = jnp.dot(agg, wrel_ref[...], preferred_element_type=jnp.float32)
    h = jnp.maximum(conv + root_ref[...], 0.0)
    mu = jnp.dot(h, wmu_ref[...], preferred_element_type=jnp.float32)
    out_ref[...] = jnp.maximum(mu + bmu_ref[...], 0.0)


_dense = pl.pallas_call(
    _dense_body,
    grid=(N // ROWS_PER_BLK,),
    in_specs=[
        pl.BlockSpec((2, ROWS_PER_BLK, D), lambda i: (0, i, 0)),
        pl.BlockSpec((ROWS_PER_BLK, D), lambda i: (i, 0)),
        pl.BlockSpec((D, D), lambda i: (0, 0)),
        pl.BlockSpec((D, D), lambda i: (0, 0)),
        pl.BlockSpec((1, D), lambda i: (0, 0)),
    ],
    out_specs=pl.BlockSpec((ROWS_PER_BLK, D), lambda i: (i, 0)),
    out_shape=jax.ShapeDtypeStruct((N, D), jnp.float32),
)


def kernel(x, message_edge_index, target_edge_index, target_edge_weights,
           W_rel, b_rel, W_root, W_mu, b_mu, W_std, b_std):
    root = _root(x, W_root, b_rel.reshape(1, D))
    agg2 = _sc_scatter(message_edge_index[0], message_edge_index[1],
                       target_edge_index[1], target_edge_index[0],
                       target_edge_weights, x)
    out = _dense(agg2, root, W_rel, W_mu, b_mu.reshape(1, D))
    return (out, target_edge_weights)


# speedup vs baseline: 1.2203x; 1.2203x over previous
import functools

import jax
import jax.numpy as jnp
from jax import lax
from jax.experimental import pallas as pl
from jax.experimental.pallas import tpu as pltpu
from jax.experimental.pallas import tpu_sc as plsc

N = 10000
D = 128
E_MSG = 256000
E_TGT = 64000
C = 80
NTILES = 16
NPAD = 10240
ROWS_PER_TILE = NPAD // NTILES

_mesh = plsc.VectorSubcoreMesh(core_axis_name="c", subcore_axis_name="s")


def _sc_body(msg_pk, tgt_pk, x, agg_out,
             msg_ib, tgt_ib, rows_v, agg_sh, sem_i, sem_g, sem_s):
    c = lax.axis_index("c")
    s = lax.axis_index("s")
    w = s * 2 + c

    def zrow(i, carry):
        for k in range(D // 16):
            rows_v[0, i, pl.ds(k * 16, 16)] = jnp.zeros((16,), jnp.float32)
        return carry
    lax.fori_loop(0, C, zrow, 0)
    for k in range(ROWS_PER_TILE // C):
        pltpu.sync_copy(rows_v.at[0],
                        agg_sh.at[pl.ds(s * ROWS_PER_TILE + k * C, C)])
    plsc.subcore_barrier()

    def run_seg(pk, ib, n, base, weighted):
        nc = 3 if weighted else 2

        def idx_desc(j):
            slot = j % 8
            return pltpu.make_async_copy(pk.at[base + j],
                                         ib.at[pl.ds(slot * nc, nc)],
                                         sem_i.at[slot])

        def gather_desc(j):
            return pltpu.make_async_copy(x.at[ib.at[(j % 8) * nc]],
                                         rows_v.at[j % 4], sem_g.at[j % 4])

        def scatter_desc(j):
            return pltpu.make_async_copy(rows_v.at[j % 4],
                                         agg_sh.at[ib.at[(j % 8) * nc + 1]],
                                         sem_s.at[j % 4])

        def scale(j):
            rbuf = j % 4
            wrow = (j % 8) * nc + 2

            def scale_body(e, carry):
                wv = plsc.bitcast(ib[wrow, pl.ds((e // 16) * 16, 16)],
                                  jnp.float32)
                oh = lax.iota(jnp.int32, 16) == (e % 16)
                ws = jnp.broadcast_to(jnp.sum(jnp.where(oh, wv, 0.0)), (16,))
                for k in range(D // 16):
                    rows_v[rbuf, e, pl.ds(k * 16, 16)] = (
                        rows_v[rbuf, e, pl.ds(k * 16, 16)] * ws)
                return carry
            lax.fori_loop(0, C, scale_body, 0)

        for k in range(4):
            idx_desc(k).start()
        for k in range(2):
            idx_desc(k).wait()
            gather_desc(k).start()

        def body(j, carry):
            gather_desc(j).wait()

            @pl.when(j >= 2)
            def _():
                scatter_desc(j - 2).wait()

            @pl.when(j + 2 < n)
            def _():
                idx_desc(j + 2).wait()
                gather_desc(j + 2).start()

            @pl.when(j + 4 < n)
            def _():
                idx_desc(j + 4).start()

            if weighted:
                scale(j)
            scatter_desc(j).start(add=True)
            return carry

        lax.fori_loop(0, n, body, 0)
        scatter_desc(n - 2).wait()
        scatter_desc(n - 1).wait()

    run_seg(msg_pk, msg_ib, 100, w * 100, weighted=False)
    run_seg(tgt_pk, tgt_ib, 25, w * 25, weighted=True)

    plsc.subcore_barrier()
    pltpu.sync_copy(agg_sh.at[pl.ds(s * ROWS_PER_TILE, ROWS_PER_TILE)],
                    agg_out.at[c, pl.ds(s * ROWS_PER_TILE, ROWS_PER_TILE)])


_sc_scatter = functools.partial(
    pl.kernel,
    out_type=jax.ShapeDtypeStruct((2, NPAD, D), jnp.float32),
    mesh=_mesh,
    compiler_params=pltpu.CompilerParams(needs_layout_passes=False),
    scratch_types=[
        pltpu.VMEM((16, C), jnp.int32),
        pltpu.VMEM((24, C), jnp.int32),
        pltpu.VMEM((4, C, D), jnp.float32),
        pltpu.VMEM_SHARED((NPAD, D), jnp.float32),
        pltpu.SemaphoreType.DMA((8,)),
        pltpu.SemaphoreType.DMA((4,)),
        pltpu.SemaphoreType.DMA((4,)),
    ],
)(_sc_body)


ROWS_PER_BLK = 1000


def _root_body(x_ref, wroot_ref, brel_ref, root_ref):
    root_ref[...] = jnp.dot(x_ref[...], wroot_ref[...],
                            preferred_element_type=jnp.float32) + brel_ref[...]


_root = pl.pallas_call(
    _root_body,
    grid=(N // ROWS_PER_BLK,),
    in_specs=[
        pl.BlockSpec((ROWS_PER_BLK, D), lambda i: (i, 0)),
        pl.BlockSpec((D, D), lambda i: (0, 0)),
        pl.BlockSpec((1, D), lambda i: (0, 0)),
    ],
    out_specs=pl.BlockSpec((ROWS_PER_BLK, D), lambda i: (i, 0)),
    out_shape=jax.ShapeDtypeStruct((N, D), jnp.float32),
)


def _dense_body(agg_ref, root_ref, wrel_ref, wmu_ref, bmu_ref, out_ref):
    agg = agg_ref[0] + agg_ref[1]
    conv = jnp.dot(agg, wrel_ref[...], preferred_element_type=jnp.float32)
    h = jnp.maximum(conv + root_ref[...], 0.0)
    mu = jnp.dot(h, wmu_ref[...], preferred_element_type=jnp.float32)
    out_ref[...] = jnp.maximum(mu + bmu_ref[...], 0.0)


_dense = pl.pallas_call(
    _dense_body,
    grid=(N // ROWS_PER_BLK,),
    in_specs=[
        pl.BlockSpec((2, ROWS_PER_BLK, D), lambda i: (0, i, 0)),
        pl.BlockSpec((ROWS_PER_BLK, D), lambda i: (i, 0)),
        pl.BlockSpec((D, D), lambda i: (0, 0)),
        pl.BlockSpec((D, D), lambda i: (0, 0)),
        pl.BlockSpec((1, D), lambda i: (0, 0)),
    ],
    out_specs=pl.BlockSpec((ROWS_PER_BLK, D), lambda i: (i, 0)),
    out_shape=jax.ShapeDtypeStruct((N, D), jnp.float32),
)


def kernel(x, message_edge_index, target_edge_index, target_edge_weights,
           W_rel, b_rel, W_root, W_mu, b_mu, W_std, b_std):
    root = _root(x, W_root, b_rel.reshape(1, D))
    msg_pk = message_edge_index.reshape(2, E_MSG // C, C).transpose(1, 0, 2)
    tw_i = lax.bitcast_convert_type(target_edge_weights, jnp.int32)
    tgt_pk = jnp.stack([target_edge_index[1].reshape(E_TGT // C, C),
                        target_edge_index[0].reshape(E_TGT // C, C),
                        tw_i.reshape(E_TGT // C, C)], axis=1)
    agg2 = _sc_scatter(msg_pk, tgt_pk, x)
    out = _dense(agg2, root, W_rel, W_mu, b_mu.reshape(1, D))
    return (out, target_edge_weights)


# speedup vs baseline: 1.3311x; 1.0908x over previous
import functools

import jax
import jax.numpy as jnp
from jax import lax
from jax.experimental import pallas as pl
from jax.experimental.pallas import tpu as pltpu
from jax.experimental.pallas import tpu_sc as plsc

N = 10000
D = 128
E_MSG = 256000
E_TGT = 64000
C = 80
NTILES = 16
NPAD = 10240
ROWS_PER_TILE = NPAD // NTILES

_mesh = plsc.VectorSubcoreMesh(core_axis_name="c", subcore_axis_name="s")


def _sc_body(msg_src, msg_dst, tgt_src, tgt_dst, tw, x, agg_out,
             src_v, dst_v, rows_v, w_v, agg_sh, sem_i, sem_g, sem_s):
    c = lax.axis_index("c")
    s = lax.axis_index("s")
    w = s * 2 + c

    def zrow(i, carry):
        for k in range(D // 16):
            rows_v[0, i, pl.ds(k * 16, 16)] = jnp.zeros((16,), jnp.float32)
        return carry
    lax.fori_loop(0, C, zrow, 0)
    for k in range(ROWS_PER_TILE // C):
        pltpu.sync_copy(rows_v.at[0],
                        agg_sh.at[pl.ds(s * ROWS_PER_TILE + k * C, C)])
    plsc.subcore_barrier()

    def run_seg(srcs, dsts, n, base, weighted):
        def idx_descs(j):
            b = (base + j) * C
            slot = j % 8
            ds = [pltpu.make_async_copy(srcs.at[pl.ds(b, C)],
                                        src_v.at[slot], sem_i.at[slot]),
                  pltpu.make_async_copy(dsts.at[pl.ds(b, C)],
                                        dst_v.at[slot], sem_i.at[slot])]
            if weighted:
                ds.append(pltpu.make_async_copy(tw.at[pl.ds(b, C)],
                                                w_v.at[slot], sem_i.at[slot]))
            return ds

        def gather_desc(j):
            return pltpu.make_async_copy(x.at[src_v.at[j % 8]],
                                         rows_v.at[j % 4], sem_g.at[j % 4])

        def scatter_desc(j):
            return pltpu.make_async_copy(rows_v.at[j % 4],
                                         agg_sh.at[dst_v.at[j % 8]],
                                         sem_s.at[j % 4])

        def scale(j):
            rbuf = j % 4
            wbuf = j % 8

            def scale_body(e, carry):
                wv = w_v[wbuf, pl.ds((e // 16) * 16, 16)]
                oh = lax.iota(jnp.int32, 16) == (e % 16)
                ws = jnp.broadcast_to(jnp.sum(jnp.where(oh, wv, 0.0)), (16,))
                for k in range(D // 16):
                    rows_v[rbuf, e, pl.ds(k * 16, 16)] = (
                        rows_v[rbuf, e, pl.ds(k * 16, 16)] * ws)
                return carry
            lax.fori_loop(0, C, scale_body, 0)

        for k in range(4):
            for d in idx_descs(k):
                d.start()
        for k in range(2):
            for d in idx_descs(k):
                d.wait()
            gather_desc(k).start()

        def body(j, carry):
            gather_desc(j).wait()

            @pl.when(j >= 2)
            def _():
                scatter_desc(j - 2).wait()

            @pl.when(j + 2 < n)
            def _():
                for d in idx_descs(j + 2):
                    d.wait()
                gather_desc(j + 2).start()

            @pl.when(j + 4 < n)
            def _():
                for d in idx_descs(j + 4):
                    d.start()

            if weighted:
                scale(j)
            scatter_desc(j).start(add=True)
            return carry

        lax.fori_loop(0, n, body, 0)
        scatter_desc(n - 2).wait()
        scatter_desc(n - 1).wait()

    run_seg(msg_src, msg_dst, 100, w * 100, weighted=False)
    run_seg(tgt_src, tgt_dst, 25, w * 25, weighted=True)

    plsc.subcore_barrier()
    pltpu.sync_copy(agg_sh.at[pl.ds(s * ROWS_PER_TILE, ROWS_PER_TILE)],
                    agg_out.at[c, pl.ds(s * ROWS_PER_TILE, ROWS_PER_TILE)])


_sc_scatter = functools.partial(
    pl.kernel,
    out_type=jax.ShapeDtypeStruct((2, NPAD, D), jnp.float32),
    mesh=_mesh,
    compiler_params=pltpu.CompilerParams(needs_layout_passes=False),
    scratch_types=[
        pltpu.VMEM((8, C), jnp.int32),
        pltpu.VMEM((8, C), jnp.int32),
        pltpu.VMEM((4, C, D), jnp.float32),
        pltpu.VMEM((8, C), jnp.float32),
        pltpu.VMEM_SHARED((NPAD, D), jnp.float32),
        pltpu.SemaphoreType.DMA((8,)),
        pltpu.SemaphoreType.DMA((4,)),
        pltpu.SemaphoreType.DMA((4,)),
    ],
)(_sc_body)


ROWS_PER_BLK = 1000


def _root_body(x_ref, wroot_ref, brel_ref, root_ref):
    root_ref[...] = jnp.dot(x_ref[...], wroot_ref[...],
                            preferred_element_type=jnp.float32) + brel_ref[...]


_root = pl.pallas_call(
    _root_body,
    grid=(N // ROWS_PER_BLK,),
    in_specs=[
        pl.BlockSpec((ROWS_PER_BLK, D), lambda i: (i, 0)),
        pl.BlockSpec((D, D), lambda i: (0, 0)),
        pl.BlockSpec((1, D), lambda i: (0, 0)),
    ],
    out_specs=pl.BlockSpec((ROWS_PER_BLK, D), lambda i: (i, 0)),
    out_shape=jax.ShapeDtypeStruct((N, D), jnp.float32),
)


def _dense_body(agg_ref, root_ref, wrel_ref, wmu_ref, bmu_ref, out_ref):
    agg = agg_ref[0] + agg_ref[1]
    conv = jnp.dot(agg, wrel_ref[...], preferred_element_type=jnp.float32)
    h = jnp.maximum(conv + root_ref[...], 0.0)
    mu = jnp.dot(h, wmu_ref[...], preferred_element_type=jnp.float32)
    out_ref[...] = jnp.maximum(mu + bmu_ref[...], 0.0)


_dense = pl.pallas_call(
    _dense_body,
    grid=(N // ROWS_PER_BLK,),
    in_specs=[
        pl.BlockSpec((2, ROWS_PER_BLK, D), lambda i: (0, i, 0)),
        pl.BlockSpec((ROWS_PER_BLK, D), lambda i: (i, 0)),
        pl.BlockSpec((D, D), lambda i: (0, 0)),
        pl.BlockSpec((D, D), lambda i: (0, 0)),
        pl.BlockSpec((1, D), lambda i: (0, 0)),
    ],
    out_specs=pl.BlockSpec((ROWS_PER_BLK, D), lambda i: (i, 0)),
    out_shape=jax.ShapeDtypeStruct((N, D), jnp.float32),
)


def kernel(x, message_edge_index, target_edge_index, target_edge_weights,
           W_rel, b_rel, W_root, W_mu, b_mu, W_std, b_std):
    root = _root(x, W_root, b_rel.reshape(1, D))
    agg2 = _sc_scatter(message_edge_index[0], message_edge_index[1],
                       target_edge_index[1], target_edge_index[0],
                       target_edge_weights, x)
    out = _dense(agg2, root, W_rel, W_mu, b_mu.reshape(1, D))
    return (out, target_edge_weights)
